# baseline (speedup 1.0000x reference)
import functools

import jax
import jax.numpy as jnp
from jax import lax
from jax.experimental import pallas as pl
from jax.experimental.pallas import tpu as pltpu
from jax.experimental.pallas import tpu_sc as plsc

_N = 10000
_E = 320000
_F_IN = 128
_F_OUT = 32
_HID = 64

_NC = 2
_NS = 16
_NW = _NC * _NS
_CH = 128
_NCHUNKS = _E // _CH
_CPW = 80
_NCH_PAD = _NW * _CPW
_EP = _NCH_PAD * _CH
_NP = 10112
_ROWS_PT = _NP // _NS

_TB = 8192
_NBP = _EP // _TB

@functools.lru_cache(maxsize=None)
def _sc_mesh():
    return plsc.VectorSubcoreMesh(
        core_axis_name="c", subcore_axis_name="s",
        num_cores=_NC, num_subcores=_NS,
    )


def _proj_body(x_ref, wp_ref, ws_ref, projw_ref, proj_ref, skip_ref):
    x = x_ref[...]
    p = jnp.dot(x, wp_ref[...], preferred_element_type=jnp.float32)
    proj_ref[...] = p
    skip_ref[...] = jnp.dot(x, ws_ref[...], preferred_element_type=jnp.float32)
    projw_ref[...] = jnp.concatenate(
        [p, jnp.zeros((_N, 128 - _F_OUT), jnp.float32)], axis=1)


_proj_call = pl.pallas_call(
    _proj_body,
    out_shape=[
        jax.ShapeDtypeStruct((_N, 128), jnp.float32),
        jax.ShapeDtypeStruct((_N, _F_OUT), jnp.float32),
        jax.ShapeDtypeStruct((_N, _F_OUT), jnp.float32),
    ],
)


def _sc_gather_body(proj, i0, i1, i2, hi, hj, hk,
                    xb, rb, cb, six0, six1, sg0, sg1, sw0, sw1):
    wid = lax.axis_index("s") * _NC + lax.axis_index("c")
    base = wid * _CPW
    idx_refs = (i0, i1, i2)
    out_refs = (hi, hj, hk)
    ixs = (six0, six1)
    gss = (sg0, sg1)
    wss = (sw0, sw1)

    def idx_cp(ch, p, t):
        return pltpu.make_async_copy(idx_refs[t].at[ch], xb.at[p, t], ixs[p])

    def gather_cp(p, t):
        return pltpu.make_async_copy(proj.at[xb.at[p, t]], rb.at[p, t], gss[p])

    def write_cp(ch, p, t):
        return pltpu.make_async_copy(
            cb.at[p, t], out_refs[t].at[pl.ds(ch * (_CH // 4), _CH // 4), :],
            wss[p])

    def repack(p, t):
        def rstep(r, carry):
            for q in range(4):
                e = 4 * r + q
                cb[p, t, r, pl.ds(32 * q, 16)] = rb[p, t, e, pl.ds(0, 16)]
                cb[p, t, r, pl.ds(32 * q + 16, 16)] = rb[p, t, e, pl.ds(16, 16)]
            return carry

        lax.fori_loop(0, _CH // 4, rstep, 0)

    for p in (0, 1):
        for t in range(3):
            idx_cp(base + p, p, t).start()

    def step(g, carry):
        for p in (0, 1):
            j = 2 * g + p
            ch = base + j
            @pl.when(g >= 1)
            def _():
                for t in range(3):
                    write_cp(ch - 2, p, t).wait()
            for t in range(3):
                idx_cp(ch, p, t).wait()
            for t in range(3):
                gather_cp(p, t).start()
            for t in range(3):
                gather_cp(p, t).wait()
            ch2 = jnp.minimum(base + j + 2, _NCH_PAD - 1)
            for t in range(3):
                idx_cp(ch2, p, t).start()
            for t in range(3):
                repack(p, t)
            for t in range(3):
                write_cp(ch, p, t).start()
        return carry

    lax.fori_loop(0, _CPW // 2, step, 0)

    for p in (0, 1):
        for t in range(3):
            idx_cp(base, p, t).wait()
        for t in range(3):
            write_cp(base + _CPW - 2 + p, p, t).wait()


@functools.lru_cache(maxsize=None)
def _sc_gather():
    return pl.kernel(
        _sc_gather_body,
        out_type=[jax.ShapeDtypeStruct((_EP // 4, 128), jnp.float32)] * 3,
        mesh=_sc_mesh(),
        scratch_types=[
            pltpu.VMEM((2, 3, _CH), jnp.int32),
            pltpu.VMEM((2, 3, _CH, 128), jnp.float32),
            pltpu.VMEM((2, 3, _CH // 4, 128), jnp.float32),
            pltpu.SemaphoreType.DMA,
            pltpu.SemaphoreType.DMA,
            pltpu.SemaphoreType.DMA,
            pltpu.SemaphoreType.DMA,
            pltpu.SemaphoreType.DMA,
            pltpu.SemaphoreType.DMA,
        ],
    )


def _scores_body(hi_ref, hj_ref, hk_ref, vf_ref, w1a, w1b, w1c, b1, w2r, b2,
                 e1a, e1b, eb1, e2, eb2, s_ref, m_ref, np_ref):
    i = pl.program_id(0)

    def unpack(ref):
        hp = ref[...]
        return jnp.concatenate(
            [hp[:, 32 * k:32 * (k + 1)] for k in range(4)], axis=0)

    hi = unpack(hi_ref)
    hj = unpack(hj_ref)
    hk = unpack(hk_ref)
    h = jnp.dot(hi, w1a[...], preferred_element_type=jnp.float32)
    h = h + jnp.dot(hj, w1b[...], preferred_element_type=jnp.float32)
    h = h + jnp.dot(hk, w1c[...], preferred_element_type=jnp.float32)
    h = jnp.maximum(h + b1[...], 0.0)
    s = jnp.sum(h * w2r[...], axis=1, keepdims=True) + b2[...]
    s = jnp.where(s > 0, s, 0.2 * s)
    s_ref[...] = s
    sm = jnp.where(vf_ref[...] > 0, s, -3.4e38)
    bm = jnp.max(sm).reshape(1, 1)
    m_ref[...] = jnp.where(i == 0, bm, jnp.maximum(m_ref[...], bm))
    e1 = jnp.dot(hj, e1a[...], preferred_element_type=jnp.float32)
    e1 = e1 + jnp.dot(hk, e1b[...], preferred_element_type=jnp.float32)
    e1 = e1 + eb1[...]
    e1 = jnp.where(e1 > 0, e1, 0.2 * e1)
    np_ref[...] = jnp.dot(e1, e2[...], preferred_element_type=jnp.float32) + eb2[...]


_scores_call = pl.pallas_call(
    _scores_body,
    grid=(_NBP,),
    in_specs=[
        pl.BlockSpec((_TB // 4, 128), lambda i: (i, 0)),
        pl.BlockSpec((_TB // 4, 128), lambda i: (i, 0)),
        pl.BlockSpec((_TB // 4, 128), lambda i: (i, 0)),
        pl.BlockSpec((_TB, 1), lambda i: (i, 0)),
        pl.BlockSpec((_F_OUT, _HID), lambda i: (0, 0)),
        pl.BlockSpec((_F_OUT, _HID), lambda i: (0, 0)),
        pl.BlockSpec((_F_OUT, _HID), lambda i: (0, 0)),
        pl.BlockSpec((1, _HID), lambda i: (0, 0)),
        pl.BlockSpec((1, _HID), lambda i: (0, 0)),
        pl.BlockSpec((1, 1), lambda i: (0, 0)),
        pl.BlockSpec((_F_OUT, _F_OUT), lambda i: (0, 0)),
        pl.BlockSpec((_F_OUT, _F_OUT), lambda i: (0, 0)),
        pl.BlockSpec((1, _F_OUT), lambda i: (0, 0)),
        pl.BlockSpec((_F_OUT, _F_OUT), lambda i: (0, 0)),
        pl.BlockSpec((1, _F_OUT), lambda i: (0, 0)),
    ],
    out_specs=[
        pl.BlockSpec((_TB, 1), lambda i: (i, 0)),
        pl.BlockSpec((1, 1), lambda i: (0, 0)),
        pl.BlockSpec((_TB, _F_OUT), lambda i: (i, 0)),
    ],
    out_shape=[
        jax.ShapeDtypeStruct((_EP, 1), jnp.float32),
        jax.ShapeDtypeStruct((1, 1), jnp.float32),
        jax.ShapeDtypeStruct((_EP, _F_OUT), jnp.float32),
    ],
)


def _weight_body(s_ref, m_ref, np_ref, vf_ref, wn_ref, z_ref):
    i = pl.program_id(0)
    w = jnp.exp(s_ref[...] - m_ref[...]) * vf_ref[...]
    wn_ref[...] = jnp.concatenate(
        [np_ref[...] * w, jnp.zeros((_TB, 128 - _F_OUT), jnp.float32)],
        axis=1)
    bz = jnp.sum(w).reshape(1, 1)
    z_ref[...] = jnp.where(i == 0, bz, z_ref[...] + bz)


_weight_call = pl.pallas_call(
    _weight_body,
    grid=(_NBP,),
    in_specs=[
        pl.BlockSpec((_TB, 1), lambda i: (i, 0)),
        pl.BlockSpec((1, 1), lambda i: (0, 0)),
        pl.BlockSpec((_TB, _F_OUT), lambda i: (i, 0)),
        pl.BlockSpec((_TB, 1), lambda i: (i, 0)),
    ],
    out_specs=[
        pl.BlockSpec((_TB, 128), lambda i: (i, 0)),
        pl.BlockSpec((1, 1), lambda i: (0, 0)),
    ],
    out_shape=[
        jax.ShapeDtypeStruct((_EP, 128), jnp.float32),
        jax.ShapeDtypeStruct((1, 1), jnp.float32),
    ],
)


def _sc_scatter_body(wn, si, out, xi, rb, acc, sl0, sl1, ss0, ss1):
    cid = lax.axis_index("c")
    sid = lax.axis_index("s")
    wid = sid * _NC + cid
    base = wid * _CPW
    sls = (sl0, sl1)
    sss = (ss0, ss1)

    z16 = jnp.zeros((16,), jnp.float32)

    def zstep(r, carry):
        for k in range(8):
            rb[0, r, pl.ds(16 * k, 16)] = z16
        return carry

    lax.fori_loop(0, _CH, zstep, 0)
    abase = sid * _ROWS_PT
    for k in range(4):
        pltpu.sync_copy(rb.at[0], acc.at[pl.ds(abase + 128 * k, 128), :])
    pltpu.sync_copy(rb.at[0, pl.ds(0, 120), :],
                    acc.at[pl.ds(abase + 512, 120), :])
    plsc.subcore_barrier()

    pltpu.sync_copy(si.at[pl.ds(base, _CPW), :], xi)

    def load_cp(j, p):
        return pltpu.make_async_copy(
            wn.at[pl.ds((base + j) * _CH, _CH), :], rb.at[p], sls[p])

    def scat_cp(j, p):
        return pltpu.make_async_copy(rb.at[p], acc.at[xi.at[j]], sss[p])

    load_cp(0, 0).start()

    def step(g, carry):
        for p in (0, 1):
            j = 2 * g + p
            load_cp(j, p).wait()
            pltpu.async_copy(rb.at[p], acc.at[xi.at[j]], sss[p], add=True)
            if p == 0:
                @pl.when(g >= 1)
                def _():
                    scat_cp(j - 1, 1).wait()
                    load_cp(j + 1, 1).start()

                @pl.when(g == 0)
                def _():
                    load_cp(j + 1, 1).start()
            else:
                scat_cp(j - 1, 0).wait()
                jn = jnp.minimum(j + 1, _CPW - 1)
                load_cp(jn, 0).start()
        return carry

    lax.fori_loop(0, _CPW // 2, step, 0)
    load_cp(_CPW - 1, 0).wait()
    scat_cp(_CPW - 1, 1).wait()
    plsc.subcore_barrier()
    pltpu.sync_copy(acc.at[pl.ds(sid * _ROWS_PT, _ROWS_PT), :],
                    out.at[cid, pl.ds(sid * _ROWS_PT, _ROWS_PT), :])


@functools.lru_cache(maxsize=None)
def _sc_scatter():
    return pl.kernel(
        _sc_scatter_body,
        out_type=jax.ShapeDtypeStruct((_NC, _NP, 128), jnp.float32),
        mesh=_sc_mesh(),
        scratch_types=[
            pltpu.VMEM((_CPW, _CH), jnp.int32),
            pltpu.VMEM((2, _CH, 128), jnp.float32),
            pltpu.VMEM_SHARED((_NP, 128), jnp.float32),
            pltpu.SemaphoreType.DMA,
            pltpu.SemaphoreType.DMA,
            pltpu.SemaphoreType.DMA,
            pltpu.SemaphoreType.DMA,
        ],
    )


def _final_body(proj_ref, skip_ref, part_ref, z_ref, theta_ref, bias_ref,
                out_ref):
    parts = part_ref[...]
    wsum = (parts[0, :_N, :_F_OUT] + parts[1, :_N, :_F_OUT]) * (1.0 / z_ref[...])
    t = proj_ref[...] * theta_ref[...] + wsum + skip_ref[...] + bias_ref[...]
    out_ref[...] = jnp.where(t > 0, t, jnp.exp(jnp.minimum(t, 0.0)) - 1.0)


_final_call = pl.pallas_call(
    _final_body,
    out_shape=jax.ShapeDtypeStruct((_N, _F_OUT), jnp.float32),
)


def kernel(in_nodes_features, edge_index, W_proj, att_W1, att_b1, att_W2,
           att_b2, edge_W1, edge_b1, edge_W2, edge_b2, theta, bias, W_skip):
    x = in_nodes_features
    projw, proj, skip = _proj_call(x, W_proj, W_skip)

    pad = _EP - _E
    padv = jnp.broadcast_to(jnp.arange(pad, dtype=jnp.int32) % _N, (3, pad))
    idxp = jnp.concatenate([edge_index, padv], axis=1)
    pos = jnp.arange(_EP, dtype=jnp.int32)
    blk = pos // _TB
    lp = pos % _TB
    qq = lp // (_TB // 4)
    rem = lp % (_TB // 4)
    perm = blk * _TB + 128 * (rem // 32) + 4 * (rem % 32) + qq
    idxp = idxp[:, perm]
    validf = (perm < _E).astype(jnp.float32).reshape(_EP, 1)
    i0 = idxp[0].reshape(_NCH_PAD, _CH)
    i1 = idxp[1].reshape(_NCH_PAD, _CH)
    i2 = idxp[2].reshape(_NCH_PAD, _CH)

    hi, hj, hk = _sc_gather()(projw, i0, i1, i2)

    s, m, nprod = _scores_call(
        hi, hj, hk, validf,
        att_W1[0:_F_OUT], att_W1[_F_OUT:2 * _F_OUT], att_W1[2 * _F_OUT:],
        att_b1.reshape(1, _HID), att_W2.reshape(1, _HID),
        att_b2.reshape(1, 1),
        edge_W1[0:_F_OUT], edge_W1[_F_OUT:],
        edge_b1.reshape(1, _F_OUT), edge_W2, edge_b2.reshape(1, _F_OUT),
    )

    wn, z = _weight_call(s, m, nprod, validf)

    partials = _sc_scatter()(wn, i0)

    out = _final_call(proj, skip, partials, z, theta.reshape(1, _F_OUT),
                      bias.reshape(1, _F_OUT))
    return out


# speedup vs baseline: 1.8052x; 1.8052x over previous
import functools

import jax
import jax.numpy as jnp
from jax import lax
from jax.experimental import pallas as pl
from jax.experimental.pallas import tpu as pltpu
from jax.experimental.pallas import tpu_sc as plsc

_N = 10000
_E = 320000
_F_IN = 128
_F_OUT = 32
_HID = 64

_NC = 2
_NS = 16
_NW = _NC * _NS
_CH = 128
_NCHUNKS = _E // _CH
_CPW = 80
_NCH_PAD = _NW * _CPW
_EP = _NCH_PAD * _CH
_NP = 10112
_ROWS_PT = _NP // _NS

_TB = 8192
_NBP = _EP // _TB

@functools.lru_cache(maxsize=None)
def _sc_mesh():
    return plsc.VectorSubcoreMesh(
        core_axis_name="c", subcore_axis_name="s",
        num_cores=_NC, num_subcores=_NS,
    )


def _proj_body(x_ref, wp_ref, ws_ref, projw_ref, proj_ref, skip_ref):
    x = x_ref[...]
    p = jnp.dot(x, wp_ref[...], preferred_element_type=jnp.float32)
    proj_ref[...] = p
    skip_ref[...] = jnp.dot(x, ws_ref[...], preferred_element_type=jnp.float32)
    projw_ref[...] = jnp.concatenate(
        [p, jnp.zeros((_N, 128 - _F_OUT), jnp.float32)], axis=1)


_proj_call = pl.pallas_call(
    _proj_body,
    out_shape=[
        jax.ShapeDtypeStruct((_N, 128), jnp.float32),
        jax.ShapeDtypeStruct((_N, _F_OUT), jnp.float32),
        jax.ShapeDtypeStruct((_N, _F_OUT), jnp.float32),
    ],
)


def _sc_gather_body(proj, i0, i1, i2, hijk,
                    xb, rb, six0, six1, sg0, sg1, sw0, sw1):
    wid = lax.axis_index("s") * _NC + lax.axis_index("c")
    base = wid * _CPW
    idx_refs = (i0, i1, i2)
    ixs = (six0, six1)
    gss = (sg0, sg1)
    wss = (sw0, sw1)

    def idx_cp(ch, p, t):
        return pltpu.make_async_copy(idx_refs[t].at[ch], xb.at[p, t], ixs[p])

    def gather_cp(p, t):
        return pltpu.make_async_copy(proj.at[xb.at[p, t]], rb.at[p, t], gss[p])

    def write_cp(ch, p):
        return pltpu.make_async_copy(
            rb.at[p, 0], hijk.at[pl.ds(ch * _CH, _CH), :], wss[p])

    for p in (0, 1):
        for t in range(3):
            idx_cp(base + p, p, t).start()

    def step(g, carry):
        for p in (0, 1):
            j = 2 * g + p
            ch = base + j
            @pl.when(g >= 1)
            def _():
                write_cp(ch - 2, p).wait()
            for t in range(3):
                idx_cp(ch, p, t).wait()
            for t in range(3):
                gather_cp(p, t).start()
            for t in range(3):
                gather_cp(p, t).wait()
            ch2 = jnp.minimum(base + j + 2, _NCH_PAD - 1)
            for t in range(3):
                idx_cp(ch2, p, t).start()
            def pack_row(r, c):
                for t in (1, 2):
                    for k in (0, 1):
                        rb[p, 0, r, pl.ds(t * _F_OUT + 16 * k, 16)] = (
                            rb[p, t, r, pl.ds(16 * k, 16)])
                return c
            lax.fori_loop(0, _CH, pack_row, 0)
            write_cp(ch, p).start()
        return carry

    lax.fori_loop(0, _CPW // 2, step, 0)

    for p in (0, 1):
        for t in range(3):
            idx_cp(base, p, t).wait()
        write_cp(base + _CPW - 2 + p, p).wait()


@functools.lru_cache(maxsize=None)
def _sc_gather():
    return pl.kernel(
        _sc_gather_body,
        out_type=jax.ShapeDtypeStruct((_EP, 128), jnp.float32),
        mesh=_sc_mesh(),
        scratch_types=[
            pltpu.VMEM((2, 3, _CH), jnp.int32),
            pltpu.VMEM((2, 3, _CH, 128), jnp.float32),
            pltpu.SemaphoreType.DMA,
            pltpu.SemaphoreType.DMA,
            pltpu.SemaphoreType.DMA,
            pltpu.SemaphoreType.DMA,
            pltpu.SemaphoreType.DMA,
            pltpu.SemaphoreType.DMA,
        ],
    )


def _scores_body(hijk_ref, w1a, w1b, w1c, b1, w2r, b2,
                 e1a, e1b, eb1, e2, eb2, s_ref, m_ref, np_ref):
    i = pl.program_id(0)
    hijk = hijk_ref[...]
    hi = hijk[:, :_F_OUT]
    hj = hijk[:, _F_OUT:2 * _F_OUT]
    hk = hijk[:, 2 * _F_OUT:3 * _F_OUT]
    h = jnp.dot(hi, w1a[...], preferred_element_type=jnp.float32)
    h = h + jnp.dot(hj, w1b[...], preferred_element_type=jnp.float32)
    h = h + jnp.dot(hk, w1c[...], preferred_element_type=jnp.float32)
    h = jnp.maximum(h + b1[...], 0.0)
    s = jnp.sum(h * w2r[...], axis=1, keepdims=True) + b2[...]
    s = jnp.where(s > 0, s, 0.2 * s)
    s_ref[...] = s
    row = lax.broadcasted_iota(jnp.int32, (_TB, 1), 0) + i * _TB
    sm = jnp.where(row < _E, s, -3.4e38)
    bm = jnp.max(sm).reshape(1, 1)
    m_ref[...] = jnp.where(i == 0, bm, jnp.maximum(m_ref[...], bm))
    e1 = jnp.dot(hj, e1a[...], preferred_element_type=jnp.float32)
    e1 = e1 + jnp.dot(hk, e1b[...], preferred_element_type=jnp.float32)
    e1 = e1 + eb1[...]
    e1 = jnp.where(e1 > 0, e1, 0.2 * e1)
    np_ref[...] = jnp.dot(e1, e2[...], preferred_element_type=jnp.float32) + eb2[...]


_scores_call = pl.pallas_call(
    _scores_body,
    grid=(_NBP,),
    in_specs=[
        pl.BlockSpec((_TB, 128), lambda i: (i, 0)),
        pl.BlockSpec((_F_OUT, _HID), lambda i: (0, 0)),
        pl.BlockSpec((_F_OUT, _HID), lambda i: (0, 0)),
        pl.BlockSpec((_F_OUT, _HID), lambda i: (0, 0)),
        pl.BlockSpec((1, _HID), lambda i: (0, 0)),
        pl.BlockSpec((1, _HID), lambda i: (0, 0)),
        pl.BlockSpec((1, 1), lambda i: (0, 0)),
        pl.BlockSpec((_F_OUT, _F_OUT), lambda i: (0, 0)),
        pl.BlockSpec((_F_OUT, _F_OUT), lambda i: (0, 0)),
        pl.BlockSpec((1, _F_OUT), lambda i: (0, 0)),
        pl.BlockSpec((_F_OUT, _F_OUT), lambda i: (0, 0)),
        pl.BlockSpec((1, _F_OUT), lambda i: (0, 0)),
    ],
    out_specs=[
        pl.BlockSpec((_TB, 1), lambda i: (i, 0)),
        pl.BlockSpec((1, 1), lambda i: (0, 0)),
        pl.BlockSpec((_TB, _F_OUT), lambda i: (i, 0)),
    ],
    out_shape=[
        jax.ShapeDtypeStruct((_EP, 1), jnp.float32),
        jax.ShapeDtypeStruct((1, 1), jnp.float32),
        jax.ShapeDtypeStruct((_EP, _F_OUT), jnp.float32),
    ],
)


def _weight_body(s_ref, m_ref, np_ref, wn_ref, z_ref):
    i = pl.program_id(0)
    row = lax.broadcasted_iota(jnp.int32, (_TB, 1), 0) + i * _TB
    valid = row < _E
    w = jnp.where(valid, jnp.exp(s_ref[...] - m_ref[...]), 0.0)
    wn_ref[...] = jnp.concatenate(
        [np_ref[...] * w, jnp.zeros((_TB, 128 - _F_OUT), jnp.float32)],
        axis=1)
    bz = jnp.sum(w).reshape(1, 1)
    z_ref[...] = jnp.where(i == 0, bz, z_ref[...] + bz)


_weight_call = pl.pallas_call(
    _weight_body,
    grid=(_NBP,),
    in_specs=[
        pl.BlockSpec((_TB, 1), lambda i: (i, 0)),
        pl.BlockSpec((1, 1), lambda i: (0, 0)),
        pl.BlockSpec((_TB, _F_OUT), lambda i: (i, 0)),
    ],
    out_specs=[
        pl.BlockSpec((_TB, 128), lambda i: (i, 0)),
        pl.BlockSpec((1, 1), lambda i: (0, 0)),
    ],
    out_shape=[
        jax.ShapeDtypeStruct((_EP, 128), jnp.float32),
        jax.ShapeDtypeStruct((1, 1), jnp.float32),
    ],
)


def _sc_scatter_body(wn, si, out, xi, rb, acc, sl0, sl1, ss0, ss1):
    cid = lax.axis_index("c")
    sid = lax.axis_index("s")
    wid = sid * _NC + cid
    base = wid * _CPW
    sls = (sl0, sl1)
    sss = (ss0, ss1)

    z16 = jnp.zeros((16,), jnp.float32)

    def zstep(r, carry):
        for k in range(8):
            rb[0, r, pl.ds(16 * k, 16)] = z16
        return carry

    lax.fori_loop(0, _CH, zstep, 0)
    abase = sid * _ROWS_PT
    for k in range(4):
        pltpu.sync_copy(rb.at[0], acc.at[pl.ds(abase + 128 * k, 128), :])
    pltpu.sync_copy(rb.at[0, pl.ds(0, 120), :],
                    acc.at[pl.ds(abase + 512, 120), :])
    plsc.subcore_barrier()

    pltpu.sync_copy(si.at[pl.ds(base, _CPW), :], xi)

    def load_cp(j, p):
        return pltpu.make_async_copy(
            wn.at[pl.ds((base + j) * _CH, _CH), :], rb.at[p], sls[p])

    def scat_cp(j, p):
        return pltpu.make_async_copy(rb.at[p], acc.at[xi.at[j]], sss[p])

    load_cp(0, 0).start()

    def step(g, carry):
        for p in (0, 1):
            j = 2 * g + p
            load_cp(j, p).wait()
            pltpu.async_copy(rb.at[p], acc.at[xi.at[j]], sss[p], add=True)
            if p == 0:
                @pl.when(g >= 1)
                def _():
                    scat_cp(j - 1, 1).wait()
                    load_cp(j + 1, 1).start()

                @pl.when(g == 0)
                def _():
                    load_cp(j + 1, 1).start()
            else:
                scat_cp(j - 1, 0).wait()
                jn = jnp.minimum(j + 1, _CPW - 1)
                load_cp(jn, 0).start()
        return carry

    lax.fori_loop(0, _CPW // 2, step, 0)
    load_cp(_CPW - 1, 0).wait()
    scat_cp(_CPW - 1, 1).wait()
    plsc.subcore_barrier()
    pltpu.sync_copy(acc.at[pl.ds(sid * _ROWS_PT, _ROWS_PT), :],
                    out.at[cid, pl.ds(sid * _ROWS_PT, _ROWS_PT), :])


@functools.lru_cache(maxsize=None)
def _sc_scatter():
    return pl.kernel(
        _sc_scatter_body,
        out_type=jax.ShapeDtypeStruct((_NC, _NP, 128), jnp.float32),
        mesh=_sc_mesh(),
        scratch_types=[
            pltpu.VMEM((_CPW, _CH), jnp.int32),
            pltpu.VMEM((2, _CH, 128), jnp.float32),
            pltpu.VMEM_SHARED((_NP, 128), jnp.float32),
            pltpu.SemaphoreType.DMA,
            pltpu.SemaphoreType.DMA,
            pltpu.SemaphoreType.DMA,
            pltpu.SemaphoreType.DMA,
        ],
    )


def _final_body(proj_ref, skip_ref, part_ref, z_ref, theta_ref, bias_ref,
                out_ref):
    parts = part_ref[...]
    wsum = (parts[0, :_N, :_F_OUT] + parts[1, :_N, :_F_OUT]) * (1.0 / z_ref[...])
    t = proj_ref[...] * theta_ref[...] + wsum + skip_ref[...] + bias_ref[...]
    out_ref[...] = jnp.where(t > 0, t, jnp.exp(jnp.minimum(t, 0.0)) - 1.0)


_final_call = pl.pallas_call(
    _final_body,
    out_shape=jax.ShapeDtypeStruct((_N, _F_OUT), jnp.float32),
)


def kernel(in_nodes_features, edge_index, W_proj, att_W1, att_b1, att_W2,
           att_b2, edge_W1, edge_b1, edge_W2, edge_b2, theta, bias, W_skip):
    x = in_nodes_features
    projw, proj, skip = _proj_call(x, W_proj, W_skip)

    pad = _EP - _E
    padv = jnp.broadcast_to(jnp.arange(pad, dtype=jnp.int32) % _N, (3, pad))
    idxp = jnp.concatenate([edge_index, padv], axis=1)
    i0 = idxp[0].reshape(_NCH_PAD, _CH)
    i1 = idxp[1].reshape(_NCH_PAD, _CH)
    i2 = idxp[2].reshape(_NCH_PAD, _CH)

    hijk = _sc_gather()(projw, i0, i1, i2)

    s, m, nprod = _scores_call(
        hijk,
        att_W1[0:_F_OUT], att_W1[_F_OUT:2 * _F_OUT], att_W1[2 * _F_OUT:],
        att_b1.reshape(1, _HID), att_W2.reshape(1, _HID),
        att_b2.reshape(1, 1),
        edge_W1[0:_F_OUT], edge_W1[_F_OUT:],
        edge_b1.reshape(1, _F_OUT), edge_W2, edge_b2.reshape(1, _F_OUT),
    )

    wn, z = _weight_call(s, m, nprod)

    partials = _sc_scatter()(wn, i0)

    out = _final_call(proj, skip, partials, z, theta.reshape(1, _F_OUT),
                      bias.reshape(1, _F_OUT))
    return out
